# Initial kernel scaffold; baseline (speedup 1.0000x reference)
import jax
import jax.numpy as jnp
from jax.experimental import pallas as pl


def kernel(walk, table):
    raise NotImplementedError("write your pallas kernel here")


# speedup vs baseline: 12.7028x; 12.7028x over previous
import functools

import jax
import jax.numpy as jnp
from jax import lax
from jax.experimental import pallas as pl
from jax.experimental.pallas import tpu as pltpu
from jax.experimental.pallas import tpu_sc as plsc

_WINDOW = 5
_NEG = 4

_NC = 2
_NS = 16
_NW = _NC * _NS


def _sc_gather(table, walk_idx, neg_idx):
    n_walk = walk_idx.shape[0]
    n_neg = neg_idx.shape[0]
    d = table.shape[1]

    walk_per_w = n_walk // _NW
    neg_per_w = n_neg // _NW
    ch_w = 640
    ch_n = 736
    assert walk_per_w % ch_w == 0 and neg_per_w % ch_n == 0
    nch_w = walk_per_w // ch_w
    nch_n = neg_per_w // ch_n

    mesh = plsc.VectorSubcoreMesh(
        core_axis_name="c", subcore_axis_name="s",
        num_cores=_NC, num_subcores=_NS)

    @functools.partial(
        pl.kernel,
        out_type=(
            jax.ShapeDtypeStruct((n_walk, d), jnp.float32),
            jax.ShapeDtypeStruct((n_neg, d), jnp.float32),
        ),
        mesh=mesh,
        scratch_types=[
            pltpu.VMEM((ch_w,), jnp.int32),
            pltpu.VMEM((ch_w, d), jnp.float32),
            pltpu.VMEM((ch_n,), jnp.int32),
            pltpu.VMEM((ch_n, d), jnp.float32),
            pltpu.SemaphoreType.DMA,
        ],
        compiler_params=pltpu.CompilerParams(use_tc_tiling_on_sc=False),
    )
    def gather_kernel(table_hbm, widx_hbm, nidx_hbm, wout_hbm, nout_hbm,
                      widx_v, wrows_v, nidx_v, nrows_v, sem):
        wid = lax.axis_index("s") * _NC + lax.axis_index("c")

        wbase = wid * walk_per_w

        def walk_chunk(c, carry):
            off = pl.multiple_of(wbase + c * ch_w, 8)
            pltpu.sync_copy(widx_hbm.at[pl.ds(off, ch_w)], widx_v)
            pltpu.async_copy(table_hbm.at[widx_v], wrows_v, sem).wait()
            pltpu.sync_copy(wrows_v, wout_hbm.at[pl.ds(off, ch_w)])
            return carry

        lax.fori_loop(0, nch_w, walk_chunk, 0)

        nbase = wid * neg_per_w

        def neg_chunk(c, carry):
            off = pl.multiple_of(nbase + c * ch_n, 8)
            pltpu.sync_copy(nidx_hbm.at[pl.ds(off, ch_n)], nidx_v)
            pltpu.async_copy(table_hbm.at[nidx_v], nrows_v, sem).wait()
            pltpu.sync_copy(nrows_v, nout_hbm.at[pl.ds(off, ch_n)])
            return carry

        lax.fori_loop(0, nch_n, neg_chunk, 0)

    return gather_kernel(table, walk_idx, neg_idx)


def _loss_body(w_ref, n_ref, out_ref, *, t, r, nb, scale):
    pid = pl.program_id(0)
    d = w_ref.shape[-1]
    w = w_ref[...]
    a2 = w[:, :t, :].reshape(r * t, d)
    acc = jnp.float32(0.0)
    for k in range(1, _WINDOW):
        p2 = w[:, k:t + k, :].reshape(r * t, d)
        s = jnp.sum(a2 * p2, axis=-1)
        acc += jnp.sum(jnp.maximum(-s, 0.0) + jnp.log1p(jnp.exp(-jnp.abs(s))))
    for k in range(_NEG):
        n2 = n_ref[k]
        nl = jnp.sum(a2 * n2, axis=-1)
        acc += jnp.sum(jnp.maximum(nl, 0.0) + jnp.log1p(jnp.exp(-jnp.abs(nl))))

    @pl.when(pid == 0)
    def _():
        out_ref[...] = jnp.zeros_like(out_ref)

    out_ref[...] += acc.reshape(1, 1)

    @pl.when(pid == nb - 1)
    def _():
        out_ref[...] *= jnp.float32(scale)


def _tc_loss(walk_e, neg_e, t):
    b, l, d = walk_e.shape
    r = 64
    nb = b // r
    n_terms = b * t * (_WINDOW - 1 + _NEG)
    body = functools.partial(_loss_body, t=t, r=r, nb=nb,
                             scale=1.0 / float(n_terms))
    out = pl.pallas_call(
        body,
        grid=(nb,),
        in_specs=[
            pl.BlockSpec((r, l, d), lambda i: (i, 0, 0)),
            pl.BlockSpec((_NEG, r * t, d), lambda i: (0, i, 0)),
        ],
        out_specs=pl.BlockSpec((1, 1), lambda i: (0, 0)),
        out_shape=jax.ShapeDtypeStruct((1, 1), jnp.float32),
    )(walk_e, neg_e)
    return out[0, 0]


def kernel(walk, table):
    b, l = walk.shape
    t = l - _WINDOW + 1
    n_nodes = table.shape[0]
    neg = jax.random.randint(jax.random.key(42), (b * t, _NEG), 1, n_nodes - 1,
                             dtype=jnp.int32)
    walk_e, neg_e = _sc_gather(table, walk.reshape(-1), neg.T.reshape(-1))
    d = table.shape[1]
    return _tc_loss(walk_e.reshape(b, l, d), neg_e.reshape(_NEG, b * t, d), t)


# speedup vs baseline: 15.9792x; 1.2579x over previous
import functools

import jax
import jax.numpy as jnp
from jax import lax
from jax.experimental import pallas as pl
from jax.experimental.pallas import tpu as pltpu
from jax.experimental.pallas import tpu_sc as plsc

_WINDOW = 5
_NEG = 4

_NC = 2
_NS = 16
_NW = _NC * _NS


def _sc_gather(table, walk_idx, neg_idx):
    n_walk = walk_idx.shape[0]
    n_neg = neg_idx.shape[0]
    d = table.shape[1]

    walk_per_w = n_walk // _NW
    neg_per_w = n_neg // _NW
    ch_w = 640
    ch_n = 736
    assert walk_per_w % ch_w == 0 and neg_per_w % ch_n == 0
    nch_w = walk_per_w // ch_w
    nch_n = neg_per_w // ch_n

    mesh = plsc.VectorSubcoreMesh(
        core_axis_name="c", subcore_axis_name="s",
        num_cores=_NC, num_subcores=_NS)

    @functools.partial(
        pl.kernel,
        out_type=(
            jax.ShapeDtypeStruct((n_walk, d), jnp.float32),
            jax.ShapeDtypeStruct((n_neg, d), jnp.float32),
        ),
        mesh=mesh,
        scratch_types=[
            pltpu.VMEM((ch_w,), jnp.int32),
            pltpu.VMEM((ch_w, d), jnp.float32),
            pltpu.VMEM((ch_n,), jnp.int32),
            pltpu.VMEM((ch_n, d), jnp.float32),
            pltpu.SemaphoreType.DMA,
        ],
        compiler_params=pltpu.CompilerParams(use_tc_tiling_on_sc=False),
    )
    def gather_kernel(table_hbm, widx_hbm, nidx_hbm, wout_hbm, nout_hbm,
                      widx_v, wrows_v, nidx_v, nrows_v, sem):
        wid = lax.axis_index("s") * _NC + lax.axis_index("c")

        wbase = wid * walk_per_w

        def walk_chunk(c, carry):
            off = pl.multiple_of(wbase + c * ch_w, 8)
            pltpu.sync_copy(widx_hbm.at[pl.ds(off, ch_w)], widx_v)
            pltpu.async_copy(table_hbm.at[widx_v], wrows_v, sem).wait()
            pltpu.sync_copy(wrows_v, wout_hbm.at[pl.ds(off, ch_w)])
            return carry

        lax.fori_loop(0, nch_w, walk_chunk, 0)

        nbase = wid * neg_per_w

        def neg_chunk(c, carry):
            off = pl.multiple_of(nbase + c * ch_n, 8)
            pltpu.sync_copy(nidx_hbm.at[pl.ds(off, ch_n)], nidx_v)
            pltpu.async_copy(table_hbm.at[nidx_v], nrows_v, sem).wait()
            pltpu.sync_copy(nrows_v, nout_hbm.at[pl.ds(off, ch_n)])
            return carry

        lax.fori_loop(0, nch_n, neg_chunk, 0)

    return gather_kernel(table, walk_idx, neg_idx)


def _rowsum(prod, ones_row):
    return lax.dot_general(ones_row, prod, (((1,), (1,)), ((), ())),
                           preferred_element_type=jnp.float32)


def _loss_body(w_ref, n_ref, out_ref, *, t, r, nb, scale):
    pid = pl.program_id(0)
    d = w_ref.shape[-1]
    w = w_ref[...]
    a2 = w[:, :t, :].reshape(r * t, d)
    ones_row = jnp.ones((1, d), jnp.float32)
    acc = jnp.float32(0.0)
    for k in range(1, _WINDOW):
        p2 = w[:, k:t + k, :].reshape(r * t, d)
        s = _rowsum(a2 * p2, ones_row)
        acc += jnp.sum(jnp.maximum(-s, 0.0) + jnp.log1p(jnp.exp(-jnp.abs(s))))
    for k in range(_NEG):
        n2 = n_ref[k]
        nl = _rowsum(a2 * n2, ones_row)
        acc += jnp.sum(jnp.maximum(nl, 0.0) + jnp.log1p(jnp.exp(-jnp.abs(nl))))

    @pl.when(pid == 0)
    def _():
        out_ref[...] = jnp.zeros_like(out_ref)

    out_ref[...] += acc.reshape(1, 1)

    @pl.when(pid == nb - 1)
    def _():
        out_ref[...] *= jnp.float32(scale)


def _tc_loss(walk_e, neg_e, t):
    b, l, d = walk_e.shape
    r = 64
    nb = b // r
    n_terms = b * t * (_WINDOW - 1 + _NEG)
    body = functools.partial(_loss_body, t=t, r=r, nb=nb,
                             scale=1.0 / float(n_terms))
    out = pl.pallas_call(
        body,
        grid=(nb,),
        in_specs=[
            pl.BlockSpec((r, l, d), lambda i: (i, 0, 0)),
            pl.BlockSpec((_NEG, r * t, d), lambda i: (0, i, 0)),
        ],
        out_specs=pl.BlockSpec((1, 1), lambda i: (0, 0)),
        out_shape=jax.ShapeDtypeStruct((1, 1), jnp.float32),
    )(walk_e, neg_e)
    return out[0, 0]


def kernel(walk, table):
    b, l = walk.shape
    t = l - _WINDOW + 1
    n_nodes = table.shape[0]
    neg = jax.random.randint(jax.random.key(42), (b * t, _NEG), 1, n_nodes - 1,
                             dtype=jnp.int32)
    walk_e, neg_e = _sc_gather(table, walk.reshape(-1), neg.T.reshape(-1))
    d = table.shape[1]
    return _tc_loss(walk_e.reshape(b, l, d), neg_e.reshape(_NEG, b * t, d), t)
